# baseline (speedup 1.0000x reference)
import functools

import jax
import jax.numpy as jnp
from jax.experimental import pallas as pl
from jax.experimental.pallas import tpu as pltpu
from jax.experimental.pallas import tpu_sc as plsc

_WINDOW = 512
_BB2 = 512


def _gather_sc(table, idx, n, embed):
    mesh = plsc.VectorSubcoreMesh(core_axis_name="c", subcore_axis_name="s")

    @functools.partial(
        pl.kernel,
        out_type=jax.ShapeDtypeStruct((n, embed), table.dtype),
        mesh=mesh,
        compiler_params=pltpu.CompilerParams(use_tc_tiling_on_sc=False),
    )
    def gather_kernel(table_hbm, i_hbm, o_hbm):
        def body(i_vmem, o_vmem):
            pltpu.sync_copy(table_hbm.at[i_vmem.at[0]], o_vmem)

        pltpu.emit_pipeline(
            body,
            grid=(n // _WINDOW,),
            in_specs=[
                pl.BlockSpec((1, _WINDOW), index_map=lambda i: (0, i))
            ],
            out_specs=[
                pl.BlockSpec((_WINDOW, embed), index_map=lambda i: (i, 0))
            ],
            core_axis_name=("c", "s"),
            dimension_semantics=(pltpu.PARALLEL,),
        )(i_hbm, o_hbm)

    return gather_kernel(table, idx)


def _transpose_tc(out_cm, hist, batch, embed):
    pair = 128 // embed
    b2 = batch // pair
    nj = b2 // _BB2
    q = out_cm.reshape(hist * b2, 128)

    def body(q_ref, o_ref):
        x = q_ref[...]
        t0 = x[:, :embed].T
        t1 = x[:, embed:].T
        o_ref[...] = jnp.stack([t0, t1], axis=2).reshape(embed, pair * _BB2)

    return pl.pallas_call(
        body,
        grid=(hist, nj),
        in_specs=[
            pl.BlockSpec((_BB2, 128), lambda h, j: (h * nj + j, 0))
        ],
        out_specs=pl.BlockSpec((embed, pair * _BB2), lambda h, j: (h, j)),
        out_shape=jax.ShapeDtypeStruct((hist * embed, batch), jnp.float32),
        compiler_params=pltpu.CompilerParams(
            dimension_semantics=("parallel", "parallel")
        ),
    )(q)


def kernel(x, table):
    batch, hist = x.shape
    n = batch * hist
    embed = table.shape[1]
    idx = x.T.reshape(1, n).astype(jnp.int32)
    out_cm = _gather_sc(table, idx, n, embed)
    out_t = _transpose_tc(out_cm, hist, batch, embed)
    return out_t.reshape(hist, embed, batch).transpose(2, 0, 1)


# speedup vs baseline: 10.2793x; 10.2793x over previous
import functools

import jax
import jax.numpy as jnp
from jax import lax
from jax.experimental import pallas as pl
from jax.experimental.pallas import tpu as pltpu
from jax.experimental.pallas import tpu_sc as plsc

_W2 = 256
_NWORKERS = 32


def kernel(x, table):
    batch, hist = x.shape
    n = batch * hist
    embed = table.shape[1]
    w = 2 * _W2
    idx = (
        x.T.reshape(n // w, _W2, 2)
        .transpose(0, 2, 1)
        .reshape(1, n)
        .astype(jnp.int32)
    )
    mesh = plsc.VectorSubcoreMesh(core_axis_name="c", subcore_axis_name="s")
    rows_per_worker = (n // 2) // _NWORKERS
    nwin = rows_per_worker // _W2

    @functools.partial(
        pl.kernel,
        out_type=jax.ShapeDtypeStruct((n // 2, 2 * embed), table.dtype),
        mesh=mesh,
        scratch_types=[
            pltpu.VMEM((w,), jnp.int32),
            pltpu.VMEM((_W2, embed), table.dtype),
            pltpu.VMEM((_W2, embed), table.dtype),
        ],
        compiler_params=pltpu.CompilerParams(use_tc_tiling_on_sc=False),
    )
    def gather_kernel(table_hbm, i_hbm, o_hbm, idx_v, rows0, rows1):
        wid = lax.axis_index("s") * 2 + lax.axis_index("c")
        base = wid * rows_per_worker

        @pl.loop(0, nwin)
        def _(win):
            r0 = base + win * _W2
            pltpu.sync_copy(i_hbm.at[0, pl.ds(2 * r0, w)], idx_v)
            pltpu.sync_copy(table_hbm.at[idx_v.at[pl.ds(0, _W2)]], rows0)
            pltpu.sync_copy(table_hbm.at[idx_v.at[pl.ds(_W2, _W2)]], rows1)
            pltpu.sync_copy(
                rows0, o_hbm.at[pl.ds(r0, _W2), pl.ds(0, embed)]
            )
            pltpu.sync_copy(
                rows1, o_hbm.at[pl.ds(r0, _W2), pl.ds(embed, embed)]
            )

    q = gather_kernel(table, idx)
    return q.reshape(hist, batch, embed).transpose(1, 0, 2)


# speedup vs baseline: 11.0804x; 1.0779x over previous
import functools

import jax
import jax.numpy as jnp
from jax import lax
from jax.experimental import pallas as pl
from jax.experimental.pallas import tpu as pltpu
from jax.experimental.pallas import tpu_sc as plsc

_W2 = 256
_NWORKERS = 32


def kernel(x, table):
    batch, hist = x.shape
    n = batch * hist
    embed = table.shape[1]
    w = 2 * _W2
    idx = (
        x.T.reshape(n // w, _W2, 2)
        .transpose(0, 2, 1)
        .reshape(1, n)
        .astype(jnp.int32)
    )
    mesh = plsc.VectorSubcoreMesh(core_axis_name="c", subcore_axis_name="s")
    rows_per_worker = (n // 2) // _NWORKERS
    nwin = rows_per_worker // _W2
    npairs = nwin // 2

    @functools.partial(
        pl.kernel,
        out_type=jax.ShapeDtypeStruct((n // 2, 2 * embed), table.dtype),
        mesh=mesh,
        scratch_types=[
            pltpu.VMEM((2, w), jnp.int32),
            pltpu.VMEM((2, _W2, embed), table.dtype),
            pltpu.VMEM((2, _W2, embed), table.dtype),
            pltpu.SemaphoreType.DMA((2,)),
            pltpu.SemaphoreType.DMA((2,)),
            pltpu.SemaphoreType.DMA((2,)),
        ],
        compiler_params=pltpu.CompilerParams(use_tc_tiling_on_sc=False),
    )
    def gather_kernel(
        table_hbm, i_hbm, o_hbm, idx_v, rows0, rows1, gsem, wsem, isem
    ):
        wid = lax.axis_index("s") * 2 + lax.axis_index("c")
        base = wid * rows_per_worker

        def r_of(win):
            return base + win * _W2

        def start_idx(win, b):
            pltpu.async_copy(
                i_hbm.at[0, pl.ds(2 * r_of(win), w)], idx_v.at[b], isem.at[b]
            )

        def wait_idx(b):
            pltpu.make_async_copy(
                i_hbm.at[0, pl.ds(0, w)], idx_v.at[b], isem.at[b]
            ).wait()

        def start_gathers(b):
            pltpu.async_copy(
                table_hbm.at[idx_v.at[b, pl.ds(0, _W2)]],
                rows0.at[b],
                gsem.at[b],
            )
            pltpu.async_copy(
                table_hbm.at[idx_v.at[b, pl.ds(_W2, _W2)]],
                rows1.at[b],
                gsem.at[b],
            )

        def wait_gathers(b):
            for r in (rows0, rows1):
                pltpu.make_async_copy(
                    table_hbm.at[idx_v.at[b, pl.ds(0, _W2)]],
                    r.at[b],
                    gsem.at[b],
                ).wait()

        def start_writes(win, b):
            pltpu.async_copy(
                rows0.at[b],
                o_hbm.at[pl.ds(r_of(win), _W2), pl.ds(0, embed)],
                wsem.at[b],
            )
            pltpu.async_copy(
                rows1.at[b],
                o_hbm.at[pl.ds(r_of(win), _W2), pl.ds(embed, embed)],
                wsem.at[b],
            )

        def wait_writes(b):
            for r in (rows0, rows1):
                pltpu.make_async_copy(
                    r.at[b],
                    o_hbm.at[pl.ds(0, _W2), pl.ds(0, embed)],
                    wsem.at[b],
                ).wait()

        start_idx(0, 0)
        wait_idx(0)
        start_gathers(0)
        start_idx(1, 1)

        @pl.loop(0, npairs)
        def _(p):
            w0 = 2 * p

            wait_gathers(0)
            start_writes(w0, 0)

            @pl.when(p < npairs - 1)
            def _():
                start_idx(w0 + 2, 0)

            @pl.when(p > 0)
            def _():
                wait_writes(1)

            wait_idx(1)
            start_gathers(1)
            wait_gathers(1)
            start_writes(w0 + 1, 1)

            @pl.when(p < npairs - 1)
            def _():
                start_idx(w0 + 3, 1)
                wait_writes(0)
                wait_idx(0)
                start_gathers(0)

        wait_writes(0)
        wait_writes(1)

    q = gather_kernel(table, idx)
    return q.reshape(hist, batch, embed).transpose(1, 0, 2)


# speedup vs baseline: 14.9512x; 1.3493x over previous
import functools

import jax
import jax.numpy as jnp
from jax.experimental import pallas as pl
from jax.experimental.pallas import tpu as pltpu
from jax.experimental.pallas import tpu_sc as plsc

_WINDOW = 512


def _gather_cm(table, idx, n, embed):
    mesh = plsc.VectorSubcoreMesh(core_axis_name="c", subcore_axis_name="s")

    @functools.partial(
        pl.kernel,
        out_type=jax.ShapeDtypeStruct((n, embed), table.dtype),
        mesh=mesh,
        compiler_params=pltpu.CompilerParams(use_tc_tiling_on_sc=False),
    )
    def gather_kernel(table_hbm, i_hbm, o_hbm):
        def body(i_vmem, o_vmem):
            pltpu.sync_copy(table_hbm.at[i_vmem.at[0]], o_vmem)

        pltpu.emit_pipeline(
            body,
            grid=(n // _WINDOW,),
            in_specs=[
                pl.BlockSpec((1, _WINDOW), index_map=lambda i: (0, i))
            ],
            out_specs=[
                pl.BlockSpec((_WINDOW, embed), index_map=lambda i: (i, 0))
            ],
            core_axis_name=("c", "s"),
            dimension_semantics=(pltpu.PARALLEL,),
        )(i_hbm, o_hbm)

    return gather_kernel(table, idx)


def kernel(x, table):
    batch, hist = x.shape
    embed = table.shape[1]
    h2 = hist // 2
    xt = x.T.astype(jnp.int32)
    halves = []
    for k in range(2):
        idx_k = xt[k * h2 : (k + 1) * h2].reshape(1, h2 * batch)
        out_k = _gather_cm(table, idx_k, h2 * batch, embed)
        halves.append(out_k.reshape(h2, batch, embed).transpose(1, 0, 2))
    return jnp.concatenate(halves, axis=1)


# speedup vs baseline: 17.1424x; 1.1466x over previous
import functools

import jax
import jax.numpy as jnp
from jax.experimental import pallas as pl
from jax.experimental.pallas import tpu as pltpu
from jax.experimental.pallas import tpu_sc as plsc

_WINDOW = 512


def kernel(x, table):
    batch, hist = x.shape
    n = batch * hist
    embed = table.shape[1]
    idx = x.T.reshape(1, n).astype(jnp.int32)
    mesh = plsc.VectorSubcoreMesh(core_axis_name="c", subcore_axis_name="s")

    @functools.partial(
        pl.kernel,
        out_type=jax.ShapeDtypeStruct((n, embed), table.dtype),
        mesh=mesh,
        compiler_params=pltpu.CompilerParams(use_tc_tiling_on_sc=False),
    )
    def gather_kernel(table_hbm, i_hbm, o_hbm):
        def body(i_vmem, o_vmem):
            pltpu.sync_copy(table_hbm.at[i_vmem.at[0]], o_vmem)

        pltpu.emit_pipeline(
            body,
            grid=(n // _WINDOW,),
            in_specs=[
                pl.BlockSpec((1, _WINDOW), index_map=lambda i: (0, i))
            ],
            out_specs=[
                pl.BlockSpec((_WINDOW, embed), index_map=lambda i: (i, 0))
            ],
            core_axis_name=("c", "s"),
            dimension_semantics=(pltpu.PARALLEL,),
        )(i_hbm, o_hbm)

    out_cm = gather_kernel(table, idx)
    return out_cm.reshape(hist, batch, embed).transpose(1, 0, 2)


# speedup vs baseline: 17.2464x; 1.0061x over previous
import functools

import jax
import jax.numpy as jnp
from jax.experimental import pallas as pl
from jax.experimental.pallas import tpu as pltpu
from jax.experimental.pallas import tpu_sc as plsc

_WINDOW = 512


def kernel(x, table):
    batch, hist = x.shape
    n = batch * hist
    embed = table.shape[1]
    idx = x.T.reshape(1, n).astype(jnp.int32)
    mesh = plsc.VectorSubcoreMesh(core_axis_name="c", subcore_axis_name="s")

    sub = _WINDOW // 2

    @functools.partial(
        pl.kernel,
        out_type=jax.ShapeDtypeStruct((n, embed), table.dtype),
        mesh=mesh,
        scratch_types=[pltpu.SemaphoreType.DMA((2,))],
        compiler_params=pltpu.CompilerParams(use_tc_tiling_on_sc=False),
    )
    def gather_kernel(table_hbm, i_hbm, o_hbm, sems):
        def body(i_vmem, o_vmem):
            for j in range(2):
                pltpu.async_copy(
                    table_hbm.at[i_vmem.at[0, pl.ds(j * sub, sub)]],
                    o_vmem.at[pl.ds(j * sub, sub)],
                    sems.at[j],
                )
            for j in range(2):
                pltpu.make_async_copy(
                    table_hbm.at[i_vmem.at[0, pl.ds(j * sub, sub)]],
                    o_vmem.at[pl.ds(j * sub, sub)],
                    sems.at[j],
                ).wait()

        pltpu.emit_pipeline(
            body,
            grid=(n // _WINDOW,),
            in_specs=[
                pl.BlockSpec((1, _WINDOW), index_map=lambda i: (0, i))
            ],
            out_specs=[
                pl.BlockSpec((_WINDOW, embed), index_map=lambda i: (i, 0))
            ],
            core_axis_name=("c", "s"),
            dimension_semantics=(pltpu.PARALLEL,),
        )(i_hbm, o_hbm)

    out_cm = gather_kernel(table, idx)
    return out_cm.reshape(hist, batch, embed).transpose(1, 0, 2)
